# Initial kernel scaffold; baseline (speedup 1.0000x reference)
import jax
import jax.numpy as jnp
from jax.experimental import pallas as pl


def kernel(x_price, x_sentiment, edge_index, edge_attr, temporal_features, p_att, p_Wz, p_bz, p_Lz_W, p_Lz_b, p_Wr, p_br, p_Lr_W, p_Lr_b, p_Wh, p_bh, p_Lh_W, p_Lh_b, s_att, s_Wz, s_bz, s_Lz_W, s_Lz_b, s_Wr, s_br, s_Lr_W, s_Lr_b, s_Wh, s_bh, s_Lh_W, s_Lh_b, red_W, red_b, lstm_Wih, lstm_Whh, lstm_bih, lstm_bhh, fc1_W, fc1_b, fc2_W, fc2_b):
    raise NotImplementedError("write your pallas kernel here")


# speedup vs baseline: 6.5870x; 6.5870x over previous
import jax
import jax.numpy as jnp
from jax import lax
from jax.experimental import pallas as pl
from jax.experimental.pallas import tpu as pltpu
from jax.experimental.pallas import tpu_sc as plsc

N = 10000
E = 320000
IN_DIM = 128
OUT_DIM = 32
HID = 128
RNN_HID = 128
PERIODS = 5

NC = 2
NS = 16
NW = NC * NS
KB = 128
NB = 79
EPW = NB * KB
EP = NW * EPW
NROW = 10240
RPT = NROW // NS
NCHUNK = 10
WC = 128


def _sc_degree_body(dst_hbm, w_hbm, zeros_hbm, out_hbm,
                    dst_v, w_v, rows_v, acc_sh):
    cid = lax.axis_index("c")
    sid = lax.axis_index("s")
    wid = cid * NS + sid
    pltpu.sync_copy(zeros_hbm, acc_sh.at[pl.ds(sid * RPT, RPT)])
    plsc.subcore_barrier()

    def block(j, carry):
        base = wid * EPW + j * KB
        pltpu.sync_copy(dst_hbm.at[pl.ds(base, KB)], dst_v)
        pltpu.sync_copy(w_hbm.at[pl.ds(base, KB)], w_v)

        def group(g, c2):
            wv = w_v[pl.ds(g * 16, 16)]
            for i in range(16):
                rows_v[g * 16 + i, :] = jnp.full((16,), wv[i], jnp.float32)
            return c2
        lax.fori_loop(0, KB // 16, group, 0)
        pltpu.sync_copy(rows_v, acc_sh.at[dst_v], add=True)
        return carry
    lax.fori_loop(0, NB, block, 0)
    plsc.subcore_barrier()
    pltpu.sync_copy(acc_sh.at[pl.ds(sid * RPT, RPT)],
                    out_hbm.at[cid].at[pl.ds(sid * RPT, RPT)])


def _sc_degree(dst_pad, w_pad, zeros16):
    mesh = plsc.VectorSubcoreMesh(core_axis_name="c", subcore_axis_name="s")
    return pl.kernel(
        _sc_degree_body,
        out_type=jax.ShapeDtypeStruct((NC, NROW, 16), jnp.float32),
        mesh=mesh,
        scratch_types=[
            pltpu.VMEM((KB,), jnp.int32),
            pltpu.VMEM((KB,), jnp.float32),
            pltpu.VMEM((KB, 16), jnp.float32),
            pltpu.VMEM_SHARED((NROW, 16), jnp.float32),
        ],
    )(dst_pad, w_pad, zeros16)


def _sc_spmm_body(src_hbm, dst_hbm, w_hbm,
                  xw0, xw1, xw2, xw3, xw4, xw5, xw6, xw7, xw8, xw9,
                  zeros_hbm, out_hbm,
                  src_v, dst_v, w_v, rows_v, acc_sh, gsem):
    cid = lax.axis_index("c")
    sid = lax.axis_index("s")
    wid = cid * NS + sid
    xws = (xw0, xw1, xw2, xw3, xw4, xw5, xw6, xw7, xw8, xw9)
    for c in range(NCHUNK):
        pltpu.sync_copy(zeros_hbm, acc_sh.at[pl.ds(sid * RPT, RPT)])
        plsc.subcore_barrier()

        def block(j, carry):
            base = wid * EPW + j * KB
            pltpu.sync_copy(src_hbm.at[pl.ds(base, KB)], src_v)
            pltpu.sync_copy(dst_hbm.at[pl.ds(base, KB)], dst_v)
            pltpu.sync_copy(w_hbm.at[pl.ds(base, KB)], w_v)
            pltpu.async_copy(xws[c].at[src_v], rows_v, gsem).wait()

            def group(g, c2):
                wv = w_v[pl.ds(g * 16, 16)]
                for i in range(16):
                    ws = jnp.full((16,), wv[i], jnp.float32)
                    e = g * 16 + i
                    for jj in range(6):
                        sl = pl.ds(jj * 16, 16)
                        rows_v[e, sl] = rows_v[e, sl] * ws
                return c2
            lax.fori_loop(0, KB // 16, group, 0)
            pltpu.sync_copy(rows_v, acc_sh.at[dst_v], add=True)
            return carry
        lax.fori_loop(0, NB, block, 0)
        plsc.subcore_barrier()
        pltpu.sync_copy(acc_sh.at[pl.ds(sid * RPT, RPT)],
                        out_hbm.at[cid, c].at[pl.ds(sid * RPT, RPT)])


def _sc_spmm(src_pad, dst_pad, w_pad, xw_chunks, zeros_hbm):
    mesh = plsc.VectorSubcoreMesh(core_axis_name="c", subcore_axis_name="s")
    return pl.kernel(
        _sc_spmm_body,
        out_type=jax.ShapeDtypeStruct((NC, NCHUNK, NROW, WC), jnp.float32),
        mesh=mesh,
        scratch_types=[
            pltpu.VMEM((KB,), jnp.int32),
            pltpu.VMEM((KB,), jnp.int32),
            pltpu.VMEM((KB,), jnp.float32),
            pltpu.VMEM((KB, WC), jnp.float32),
            pltpu.VMEM_SHARED((NROW, WC), jnp.float32),
            pltpu.SemaphoreType.DMA,
        ],
    )(src_pad, dst_pad, w_pad, *xw_chunks, zeros_hbm)


def _tc1_body(x_ref, w_ref, d0_ref, d1_ref, o_ref):
    dinv = lax.rsqrt(d0_ref[...] + d1_ref[...] + 1.0)
    o_ref[0] = jnp.dot(x_ref[0], w_ref[0],
                       preferred_element_type=jnp.float32) * dinv


def _tc1(x_all, w_all, d0, d1):
    B = 400
    nb = N // B
    return pl.pallas_call(
        _tc1_body,
        grid=(NCHUNK, nb),
        in_specs=[
            pl.BlockSpec((1, B, IN_DIM), lambda q, i: (q, i, 0)),
            pl.BlockSpec((1, IN_DIM, WC), lambda q, i: (q, 0, 0)),
            pl.BlockSpec((B, 1), lambda q, i: (i, 0)),
            pl.BlockSpec((B, 1), lambda q, i: (i, 0)),
        ],
        out_specs=pl.BlockSpec((1, B, WC), lambda q, i: (q, i, 0)),
        out_shape=jax.ShapeDtypeStruct((NCHUNK, N, WC), jnp.float32),
    )(x_all, w_all, d0, d1)


def _tc2_body(z_ref, xw_ref, d0_ref, d1_ref, h0_ref, att_ref, gb_ref,
              lsel_ref, lw_ref, lb_ref, selp_ref, sels_ref,
              redw_ref, redb_ref, wih_ref, bih_ref, bhh_ref,
              o_ref):
    dinv = lax.rsqrt(d0_ref[...] + d1_ref[...] + 1.0)
    H0 = h0_ref[...]
    PS = []
    for b in range(2):
        es = [jnp.exp(att_ref[b, t]) for t in range(PERIODS)]
        tot = es[0] + es[1] + es[2] + es[3] + es[4]
        H = H0
        Hacc = jnp.zeros_like(H)
        for t in range(PERIODS):
            q = b * PERIODS + t
            ys = dinv * (z_ref[0, q] + z_ref[1, q] + xw_ref[q])
            pre = []
            for gi in range(3):
                Lg = lw_ref[b, gi]
                Hin = H if gi != 2 else H * jax.nn.sigmoid(pre[1])
                pre.append(
                    jnp.dot(ys, lsel_ref[b, gi],
                            preferred_element_type=jnp.float32)
                    + jnp.dot(Hin, Lg[32:], preferred_element_type=jnp.float32)
                    + jnp.dot(gb_ref[b, gi], Lg[:32],
                              preferred_element_type=jnp.float32)
                    + lb_ref[b, gi])
            Zg = jax.nn.sigmoid(pre[0])
            Ht = jnp.tanh(pre[2])
            H = Zg * H + (1.0 - Zg) * Ht
            Hacc = Hacc + (es[t] / tot) * H
        PS.append(Hacc)
    P, S = PS
    PB = jnp.dot(P, selp_ref[...], preferred_element_type=jnp.float32)
    SB = jnp.dot(S, sels_ref[...], preferred_element_type=jnp.float32)
    kron = PB * SB
    red = jnp.dot(kron, redw_ref[...],
                  preferred_element_type=jnp.float32) + redb_ref[...]
    o_ref[...] = (jnp.dot(red, wih_ref[...],
                          preferred_element_type=jnp.float32)
                  + bih_ref[...] + bhh_ref[...])


def _tc2(z, xw, d0, d1, h0, att, gb, lsel, lw, lb, selp, sels,
         red_W, red_b, wih, bih, bhh):
    B = 200
    nb = N // B
    return pl.pallas_call(
        _tc2_body,
        grid=(nb,),
        in_specs=[
            pl.BlockSpec((NC, NCHUNK, B, WC), lambda i: (0, 0, i, 0)),
            pl.BlockSpec((NCHUNK, B, WC), lambda i: (0, i, 0)),
            pl.BlockSpec((B, 1), lambda i: (i, 0)),
            pl.BlockSpec((B, 1), lambda i: (i, 0)),
            pl.BlockSpec((B, OUT_DIM), lambda i: (i, 0)),
            pl.BlockSpec(memory_space=pltpu.SMEM),
            pl.BlockSpec((2, 3, 1, OUT_DIM), lambda i: (0, 0, 0, 0)),
            pl.BlockSpec((2, 3, WC, OUT_DIM), lambda i: (0, 0, 0, 0)),
            pl.BlockSpec((2, 3, 2 * OUT_DIM, OUT_DIM), lambda i: (0, 0, 0, 0)),
            pl.BlockSpec((2, 3, 1, OUT_DIM), lambda i: (0, 0, 0, 0)),
            pl.BlockSpec((OUT_DIM, OUT_DIM * OUT_DIM), lambda i: (0, 0)),
            pl.BlockSpec((OUT_DIM, OUT_DIM * OUT_DIM), lambda i: (0, 0)),
            pl.BlockSpec((OUT_DIM * OUT_DIM, HID), lambda i: (0, 0)),
            pl.BlockSpec((1, HID), lambda i: (0, 0)),
            pl.BlockSpec((HID, 4 * RNN_HID), lambda i: (0, 0)),
            pl.BlockSpec((1, 4 * RNN_HID), lambda i: (0, 0)),
            pl.BlockSpec((1, 4 * RNN_HID), lambda i: (0, 0)),
        ],
        out_specs=pl.BlockSpec((B, 4 * RNN_HID), lambda i: (i, 0)),
        out_shape=jax.ShapeDtypeStruct((N, 4 * RNN_HID), jnp.float32),
    )(z, xw, d0, d1, h0, att, gb, lsel, lw, lb, selp, sels,
      red_W, red_b, wih, bih, bhh)


def _tc3_body(xp_ref, whh_ref, fc1w_ref, fc1b_ref, fc2w_ref, fc2b_ref,
              o_ref, h_s, c_s):
    pid = pl.program_id(0)

    @pl.when(pid == 0)
    def _():
        h_s[...] = jnp.zeros_like(h_s)
        c_s[...] = jnp.zeros_like(c_s)

    whh = whh_ref[...]

    def step(i, carry):
        h, c = carry
        g = xp_ref[pl.ds(i, 1), :] + jnp.dot(
            h, whh, preferred_element_type=jnp.float32)
        ii = jax.nn.sigmoid(g[:, 0:128])
        ff = jax.nn.sigmoid(g[:, 128:256])
        gg = jnp.tanh(g[:, 256:384])
        oo = jax.nn.sigmoid(g[:, 384:512])
        c = ff * c + ii * gg
        h = oo * jnp.tanh(c)
        return (h, c)

    h, c = lax.fori_loop(0, xp_ref.shape[0], step, (h_s[...], c_s[...]))
    h_s[...] = h
    c_s[...] = c

    @pl.when(pid == pl.num_programs(0) - 1)
    def _():
        hid = jax.nn.relu(jnp.dot(h, fc1w_ref[...],
                                  preferred_element_type=jnp.float32)
                          + fc1b_ref[...])
        o_ref[...] = (jnp.dot(hid, fc2w_ref[...],
                              preferred_element_type=jnp.float32)
                      + fc2b_ref[...])


def _tc3(xp, whh, fc1w, fc1b, fc2w, fc2b):
    B = 1000
    nb = N // B
    return pl.pallas_call(
        _tc3_body,
        grid=(nb,),
        in_specs=[
            pl.BlockSpec((B, 4 * RNN_HID), lambda i: (i, 0)),
            pl.BlockSpec((RNN_HID, 4 * RNN_HID), lambda i: (0, 0)),
            pl.BlockSpec((RNN_HID, HID), lambda i: (0, 0)),
            pl.BlockSpec((1, HID), lambda i: (0, 0)),
            pl.BlockSpec((HID, 1), lambda i: (0, 0)),
            pl.BlockSpec((1, 1), lambda i: (0, 0)),
        ],
        out_specs=pl.BlockSpec((1, 1), lambda i: (0, 0)),
        out_shape=jax.ShapeDtypeStruct((1, 1), jnp.float32),
        scratch_shapes=[
            pltpu.VMEM((1, RNN_HID), jnp.float32),
            pltpu.VMEM((1, RNN_HID), jnp.float32),
        ],
    )(xp, whh, fc1w, fc1b, fc2w, fc2b)


def kernel(x_price, x_sentiment, edge_index, edge_attr, temporal_features,
           p_att, p_Wz, p_bz, p_Lz_W, p_Lz_b, p_Wr, p_br, p_Lr_W, p_Lr_b,
           p_Wh, p_bh, p_Lh_W, p_Lh_b,
           s_att, s_Wz, s_bz, s_Lz_W, s_Lz_b, s_Wr, s_br, s_Lr_W, s_Lr_b,
           s_Wh, s_bh, s_Lh_W, s_Lh_b,
           red_W, red_b, lstm_Wih, lstm_Whh, lstm_bih, lstm_bhh,
           fc1_W, fc1_b, fc2_W, fc2_b):
    src = edge_index[0]
    dst = edge_index[1]
    pad = EP - E
    src_pad = jnp.concatenate([src, jnp.zeros((pad,), jnp.int32)])
    dst_pad = jnp.concatenate([dst, jnp.zeros((pad,), jnp.int32)])
    w_pad = jnp.concatenate([edge_attr, jnp.zeros((pad,), jnp.float32)])
    zeros_hbm = jnp.zeros((RPT, WC), jnp.float32)
    zeros16 = jnp.zeros((RPT, 16), jnp.float32)

    degp = _sc_degree(dst_pad, w_pad, zeros16)
    d0 = degp[0, :N, 0:1]
    d1 = degp[1, :N, 0:1]

    padw = jnp.zeros((IN_DIM, WC - 3 * OUT_DIM), jnp.float32)
    wc_p = jnp.concatenate([p_Wz, p_Wr, p_Wh, padw], axis=1)
    wc_s = jnp.concatenate([s_Wz, s_Wr, s_Wh, padw], axis=1)
    w_all = jnp.stack([wc_p] * PERIODS + [wc_s] * PERIODS)
    x_all = jnp.concatenate([x_price.transpose(2, 0, 1),
                             x_sentiment.transpose(2, 0, 1)])
    xwp = _tc1(x_all, w_all, d0, d1)
    xw_chunks = [xwp[c] for c in range(NCHUNK)]

    z = _sc_spmm(src_pad, dst_pad, w_pad, xw_chunks, zeros_hbm)

    att = jnp.stack([p_att, s_att])
    gb = jnp.stack([jnp.stack([p_bz, p_br, p_bh]),
                    jnp.stack([s_bz, s_br, s_bh])]).reshape(2, 3, 1, OUT_DIM)
    lw = jnp.stack([jnp.stack([p_Lz_W, p_Lr_W, p_Lh_W]),
                    jnp.stack([s_Lz_W, s_Lr_W, s_Lh_W])])
    lb = jnp.stack([jnp.stack([p_Lz_b, p_Lr_b, p_Lh_b]),
                    jnp.stack([s_Lz_b, s_Lr_b, s_Lh_b])]).reshape(2, 3, 1, OUT_DIM)
    lsel = jnp.zeros((2, 3, WC, OUT_DIM), jnp.float32)
    for b in range(2):
        for gi in range(3):
            lsel = lsel.at[b, gi, gi * OUT_DIM:(gi + 1) * OUT_DIM, :].set(
                lw[b, gi, :OUT_DIM])
    eye = jnp.eye(OUT_DIM, dtype=jnp.float32)
    selp = jnp.repeat(eye, OUT_DIM, axis=1)
    sels = jnp.tile(eye, (1, OUT_DIM))
    xp = _tc2(z, xwp, d0, d1, temporal_features, att, gb, lsel, lw, lb,
              selp, sels, red_W, red_b.reshape(1, HID), lstm_Wih,
              lstm_bih.reshape(1, -1), lstm_bhh.reshape(1, -1))

    out = _tc3(xp, lstm_Whh, fc1_W, fc1_b.reshape(1, HID),
               fc2_W, fc2_b.reshape(1, 1))
    return out
